# baseline (speedup 1.0000x reference)
import functools

import jax
import jax.numpy as jnp
from jax import lax
from jax.experimental import pallas as pl
from jax.experimental.pallas import tpu as pltpu
from jax.experimental.pallas import tpu_sc as plsc

GAMMA = 12.0
HIDDEN = 64
LANES = 16
NC = 2
NS = 16
NW = NC * NS


def _sc_body(heads, rels, tails, etab, rtab, wsplit, out,
             hidx, ridx, tidx, hrow, rrow, trow, wv,
             absb, dotb, outv, sem_h, sem_r, sem_t, b_per_w):
    wid = lax.axis_index("s") * NC + lax.axis_index("c")
    base = wid * b_per_w

    pltpu.sync_copy(heads.at[pl.ds(base, b_per_w)], hidx)
    pltpu.sync_copy(rels.at[pl.ds(base, b_per_w)], ridx)
    pltpu.sync_copy(tails.at[pl.ds(base, b_per_w)], tidx)
    pltpu.sync_copy(wsplit, wv)

    cp_h = pltpu.async_copy(etab.at[hidx], hrow, sem_h)
    cp_r = pltpu.async_copy(rtab.at[ridx], rrow, sem_r)
    cp_t = pltpu.async_copy(etab.at[tidx], trow, sem_t)
    cp_h.wait()
    cp_r.wait()
    cp_t.wait()

    wregs = [wv[i] for i in range(4)]
    bvec = wv[4]

    def triple_body(j, carry):
        acc_abs = jnp.zeros((LANES,), jnp.float32)
        acc_dot = jnp.zeros((LANES,), jnp.float32)
        for c in range(2):
            he, ho = plsc.unpack(
                hrow[j, pl.ds(32 * c, 32)], format=plsc.PackFormat.INTERLEAVED)
            re, ro = plsc.unpack(
                rrow[j, pl.ds(32 * c, 32)], format=plsc.PackFormat.INTERLEAVED)
            te, to = plsc.unpack(
                trow[j, pl.ds(32 * c, 32)], format=plsc.PackFormat.INTERLEAVED)
            ze = (he + re) - te
            zo = (ho + ro) - to
            acc_abs = acc_abs + jnp.abs(ze) + jnp.abs(zo)
            acc_dot = acc_dot + ze * wregs[2 * c] + zo * wregs[2 * c + 1]
        absb[j] = plsc.cumsum(acc_abs)
        dotb[j] = plsc.cumsum(acc_dot)
        return carry

    lax.fori_loop(0, b_per_w, triple_body, 0)

    riota = lax.iota(jnp.int32, LANES)
    col15 = jnp.full((LANES,), LANES - 1, jnp.int32)

    def group_body(g, carry):
        gbase = g * LANES
        rows = riota + gbase
        sa = plsc.load_gather(absb, [rows, col15])
        sd = plsc.load_gather(dotb, [rows, col15]) + bvec
        dcoef = 1.0 / (1.0 + jnp.exp(-sd))
        outv[pl.ds(gbase, LANES)] = GAMMA - dcoef * sa
        return carry

    lax.fori_loop(0, b_per_w // LANES, group_body, 0)

    pltpu.sync_copy(outv, out.at[pl.ds(base, b_per_w)])


@jax.jit
def _run(heads, rels, tails, etab, rtab, wsplit):
    batch = heads.shape[0]
    b_per_w = batch // NW
    mesh = plsc.VectorSubcoreMesh(core_axis_name="c", subcore_axis_name="s")
    kern = functools.partial(
        pl.kernel,
        out_type=jax.ShapeDtypeStruct((batch,), jnp.float32),
        mesh=mesh,
        compiler_params=pltpu.CompilerParams(
            needs_layout_passes=False, use_tc_tiling_on_sc=False),
        scratch_types=[
            pltpu.VMEM((b_per_w,), jnp.int32),
            pltpu.VMEM((b_per_w,), jnp.int32),
            pltpu.VMEM((b_per_w,), jnp.int32),
            pltpu.VMEM((b_per_w, HIDDEN), jnp.bfloat16),
            pltpu.VMEM((b_per_w, HIDDEN), jnp.bfloat16),
            pltpu.VMEM((b_per_w, HIDDEN), jnp.bfloat16),
            pltpu.VMEM((5, LANES), jnp.float32),
            pltpu.VMEM((b_per_w, LANES), jnp.float32),
            pltpu.VMEM((b_per_w, LANES), jnp.float32),
            pltpu.VMEM((b_per_w,), jnp.float32),
            pltpu.SemaphoreType.DMA,
            pltpu.SemaphoreType.DMA,
            pltpu.SemaphoreType.DMA,
        ],
    )(functools.partial(_sc_body, b_per_w=b_per_w))
    return kern(heads, rels, tails, etab, rtab, wsplit)


def kernel(sample, entity_embedding, relation_embedding, D_w, D_b):
    heads = sample[:, 0]
    rels = sample[:, 1]
    tails = sample[:, 2]
    etab = entity_embedding.astype(jnp.bfloat16)
    rtab = relation_embedding.astype(jnp.bfloat16)
    w4 = D_w[:, 0].reshape(2, LANES, 2)
    wsplit = jnp.concatenate(
        [jnp.stack([w4[c, :, 0], w4[c, :, 1]]) for c in range(2)]
        + [jnp.broadcast_to(D_b, (1, LANES))], axis=0)
    out = _run(heads, rels, tails, etab, rtab, wsplit)
    return out[:, None]


# speedup vs baseline: 1.2095x; 1.2095x over previous
import functools

import jax
import jax.numpy as jnp
from jax import lax
from jax.experimental import pallas as pl
from jax.experimental.pallas import tpu as pltpu
from jax.experimental.pallas import tpu_sc as plsc

GAMMA = 12.0
HIDDEN = 64
LANES = 16
NC = 2
NS = 16
NW = NC * NS


def _tc_txp_body(eye_ref, x_ref, o_ref):
    o_ref[...] = jax.lax.dot_general(
        x_ref[...], eye_ref[...],
        dimension_numbers=(((0,), (0,)), ((), ())),
        preferred_element_type=jnp.float32)


def _tc_transpose(table_t, n_rows, blk):
    grid = (n_rows + blk - 1) // blk
    eye = jnp.eye(HIDDEN, dtype=jnp.float32)
    return pl.pallas_call(
        _tc_txp_body,
        grid=(grid,),
        in_specs=[
            pl.BlockSpec((HIDDEN, HIDDEN), lambda i: (0, 0)),
            pl.BlockSpec((HIDDEN, blk), lambda i: (0, i)),
        ],
        out_specs=pl.BlockSpec((blk, HIDDEN), lambda i: (i, 0)),
        out_shape=jax.ShapeDtypeStruct((n_rows, HIDDEN), jnp.float32),
    )(eye, table_t)


def _sc_body(heads, rels, tails, etab, rtab, wsplit, out,
             hidx, ridx, tidx, hrow, rrow, trow, wv,
             absb, dotb, outv, sem_h, sem_r, sem_t, b_per_w):
    wid = lax.axis_index("s") * NC + lax.axis_index("c")
    base = wid * b_per_w

    pltpu.sync_copy(heads.at[pl.ds(base, b_per_w)], hidx)
    pltpu.sync_copy(rels.at[pl.ds(base, b_per_w)], ridx)
    pltpu.sync_copy(tails.at[pl.ds(base, b_per_w)], tidx)
    pltpu.sync_copy(wsplit, wv)

    cp_h = pltpu.async_copy(etab.at[hidx], hrow, sem_h)
    cp_r = pltpu.async_copy(rtab.at[ridx], rrow, sem_r)
    cp_t = pltpu.async_copy(etab.at[tidx], trow, sem_t)
    cp_h.wait()
    cp_r.wait()
    cp_t.wait()

    wregs = [wv[i] for i in range(4)]
    bvec = wv[4]

    def triple_body(j, carry):
        acc_abs = jnp.zeros((LANES,), jnp.float32)
        acc_dot = jnp.zeros((LANES,), jnp.float32)
        for c in range(4):
            hv = hrow[j, pl.ds(LANES * c, LANES)]
            rv = rrow[j, pl.ds(LANES * c, LANES)]
            tv = trow[j, pl.ds(LANES * c, LANES)]
            z = (hv + rv) - tv
            acc_abs = acc_abs + jnp.abs(z)
            acc_dot = acc_dot + z * wregs[c]
        absb[j] = plsc.cumsum(acc_abs)
        dotb[j] = plsc.cumsum(acc_dot)
        return carry

    lax.fori_loop(0, b_per_w, triple_body, 0)

    riota = lax.iota(jnp.int32, LANES)
    col15 = jnp.full((LANES,), LANES - 1, jnp.int32)

    def group_body(g, carry):
        gbase = g * LANES
        rows = riota + gbase
        sa = plsc.load_gather(absb, [rows, col15])
        sd = plsc.load_gather(dotb, [rows, col15]) + bvec
        dcoef = 1.0 / (1.0 + jnp.exp(-sd))
        outv[pl.ds(gbase, LANES)] = GAMMA - dcoef * sa
        return carry

    lax.fori_loop(0, b_per_w // LANES, group_body, 0)

    pltpu.sync_copy(outv, out.at[pl.ds(base, b_per_w)])


@jax.jit
def _run(heads, rels, tails, etab, rtab, wsplit):
    batch = heads.shape[0]
    b_per_w = batch // NW
    mesh = plsc.VectorSubcoreMesh(core_axis_name="c", subcore_axis_name="s")
    kern = functools.partial(
        pl.kernel,
        out_type=jax.ShapeDtypeStruct((batch,), jnp.float32),
        mesh=mesh,
        compiler_params=pltpu.CompilerParams(
            needs_layout_passes=False, use_tc_tiling_on_sc=False),
        scratch_types=[
            pltpu.VMEM((b_per_w,), jnp.int32),
            pltpu.VMEM((b_per_w,), jnp.int32),
            pltpu.VMEM((b_per_w,), jnp.int32),
            pltpu.VMEM((b_per_w, HIDDEN), jnp.float32),
            pltpu.VMEM((b_per_w, HIDDEN), jnp.float32),
            pltpu.VMEM((b_per_w, HIDDEN), jnp.float32),
            pltpu.VMEM((5, LANES), jnp.float32),
            pltpu.VMEM((b_per_w, LANES), jnp.float32),
            pltpu.VMEM((b_per_w, LANES), jnp.float32),
            pltpu.VMEM((b_per_w,), jnp.float32),
            pltpu.SemaphoreType.DMA,
            pltpu.SemaphoreType.DMA,
            pltpu.SemaphoreType.DMA,
        ],
    )(functools.partial(_sc_body, b_per_w=b_per_w))
    return kern(heads, rels, tails, etab, rtab, wsplit)


def kernel(sample, entity_embedding, relation_embedding, D_w, D_b):
    heads = sample[:, 0]
    rels = sample[:, 1]
    tails = sample[:, 2]
    etab = _tc_transpose(entity_embedding.T, entity_embedding.shape[0], 8192)
    rtab = _tc_transpose(relation_embedding.T, relation_embedding.shape[0], 1000)
    wsplit = jnp.concatenate(
        [D_w[:, 0].reshape(4, LANES), jnp.broadcast_to(D_b, (1, LANES))],
        axis=0)
    out = _run(heads, rels, tails, etab, rtab, wsplit)
    return out[:, None]


# speedup vs baseline: 1.2213x; 1.0098x over previous
import functools

import jax
import jax.numpy as jnp
from jax import lax
from jax.experimental import pallas as pl
from jax.experimental.pallas import tpu as pltpu
from jax.experimental.pallas import tpu_sc as plsc

GAMMA = 12.0
HIDDEN = 64
LANES = 16
NC = 2
NS = 16
NW = NC * NS


def _tc_txp_body(x_ref, o_ref):
    o_ref[...] = x_ref[...].T


def _tc_transpose(table_t, n_rows, blk):
    grid = (n_rows + blk - 1) // blk
    return pl.pallas_call(
        _tc_txp_body,
        grid=(grid,),
        in_specs=[
            pl.BlockSpec((HIDDEN, blk), lambda i: (0, i)),
        ],
        out_specs=pl.BlockSpec((blk, HIDDEN), lambda i: (i, 0)),
        out_shape=jax.ShapeDtypeStruct((n_rows, HIDDEN), jnp.float32),
    )(table_t)


def _sc_body(heads, rels, tails, etab, rtab, wsplit, out,
             hidx, ridx, tidx, hrow, rrow, trow, wv,
             absb, dotb, outv, sem_h, sem_r, sem_t, b_per_w):
    wid = lax.axis_index("s") * NC + lax.axis_index("c")
    base = wid * b_per_w

    pltpu.sync_copy(heads.at[pl.ds(base, b_per_w)], hidx)
    pltpu.sync_copy(rels.at[pl.ds(base, b_per_w)], ridx)
    pltpu.sync_copy(tails.at[pl.ds(base, b_per_w)], tidx)
    pltpu.sync_copy(wsplit, wv)

    cp_h = pltpu.async_copy(etab.at[hidx], hrow, sem_h)
    cp_r = pltpu.async_copy(rtab.at[ridx], rrow, sem_r)
    cp_t = pltpu.async_copy(etab.at[tidx], trow, sem_t)
    cp_h.wait()
    cp_r.wait()
    cp_t.wait()

    wregs = [wv[i] for i in range(4)]
    bvec = wv[4]

    def triple_body(j, carry):
        acc_abs = jnp.zeros((LANES,), jnp.float32)
        acc_dot = jnp.zeros((LANES,), jnp.float32)
        for c in range(4):
            hv = hrow[j, pl.ds(LANES * c, LANES)]
            rv = rrow[j, pl.ds(LANES * c, LANES)]
            tv = trow[j, pl.ds(LANES * c, LANES)]
            z = (hv + rv) - tv
            acc_abs = acc_abs + jnp.abs(z)
            acc_dot = acc_dot + z * wregs[c]
        absb[j] = plsc.cumsum(acc_abs)
        dotb[j] = plsc.cumsum(acc_dot)
        return carry

    lax.fori_loop(0, b_per_w, triple_body, 0)

    riota = lax.iota(jnp.int32, LANES)
    col15 = jnp.full((LANES,), LANES - 1, jnp.int32)

    def group_body(g, carry):
        gbase = g * LANES
        rows = riota + gbase
        sa = plsc.load_gather(absb, [rows, col15])
        sd = plsc.load_gather(dotb, [rows, col15]) + bvec
        dcoef = 1.0 / (1.0 + jnp.exp(-sd))
        outv[pl.ds(gbase, LANES)] = GAMMA - dcoef * sa
        return carry

    lax.fori_loop(0, b_per_w // LANES, group_body, 0)

    pltpu.sync_copy(outv, out.at[pl.ds(base, b_per_w)])


@jax.jit
def _run(heads, rels, tails, etab, rtab, wsplit):
    batch = heads.shape[0]
    b_per_w = batch // NW
    mesh = plsc.VectorSubcoreMesh(core_axis_name="c", subcore_axis_name="s")
    kern = functools.partial(
        pl.kernel,
        out_type=jax.ShapeDtypeStruct((batch,), jnp.float32),
        mesh=mesh,
        compiler_params=pltpu.CompilerParams(
            needs_layout_passes=False, use_tc_tiling_on_sc=False),
        scratch_types=[
            pltpu.VMEM((b_per_w,), jnp.int32),
            pltpu.VMEM((b_per_w,), jnp.int32),
            pltpu.VMEM((b_per_w,), jnp.int32),
            pltpu.VMEM((b_per_w, HIDDEN), jnp.float32),
            pltpu.VMEM((b_per_w, HIDDEN), jnp.float32),
            pltpu.VMEM((b_per_w, HIDDEN), jnp.float32),
            pltpu.VMEM((5, LANES), jnp.float32),
            pltpu.VMEM((b_per_w, LANES), jnp.float32),
            pltpu.VMEM((b_per_w, LANES), jnp.float32),
            pltpu.VMEM((b_per_w,), jnp.float32),
            pltpu.SemaphoreType.DMA,
            pltpu.SemaphoreType.DMA,
            pltpu.SemaphoreType.DMA,
        ],
    )(functools.partial(_sc_body, b_per_w=b_per_w))
    return kern(heads, rels, tails, etab, rtab, wsplit)


def kernel(sample, entity_embedding, relation_embedding, D_w, D_b):
    heads = sample[:, 0]
    rels = sample[:, 1]
    tails = sample[:, 2]
    etab = _tc_transpose(entity_embedding.T, entity_embedding.shape[0], 8192)
    rtab = _tc_transpose(relation_embedding.T, relation_embedding.shape[0], 1000)
    wsplit = jnp.concatenate(
        [D_w[:, 0].reshape(4, LANES), jnp.broadcast_to(D_b, (1, LANES))],
        axis=0)
    out = _run(heads, rels, tails, etab, rtab, wsplit)
    return out[:, None]


# speedup vs baseline: 1.2234x; 1.0017x over previous
import functools

import jax
import jax.numpy as jnp
from jax import lax
from jax.experimental import pallas as pl
from jax.experimental.pallas import tpu as pltpu
from jax.experimental.pallas import tpu_sc as plsc

GAMMA = 12.0
HIDDEN = 64
LANES = 16
NC = 2
NS = 16
NW = NC * NS


def _tc_txp_body(x_ref, o_ref):
    o_ref[...] = x_ref[...].T


def _tc_transpose(table_t, n_rows, blk):
    grid = (n_rows + blk - 1) // blk
    return pl.pallas_call(
        _tc_txp_body,
        grid=(grid,),
        in_specs=[
            pl.BlockSpec((HIDDEN, blk), lambda i: (0, i)),
        ],
        out_specs=pl.BlockSpec((blk, HIDDEN), lambda i: (i, 0)),
        out_shape=jax.ShapeDtypeStruct((n_rows, HIDDEN), jnp.float32),
    )(table_t)


def _sc_body(heads, rels, tails, etab, rtab, wsplit, out,
             hidx, ridx, tidx, hrow, rrow, trow, wv,
             absb, dotb, outv, sem_h, sem_r, sem_t, b_per_w):
    wid = lax.axis_index("s") * NC + lax.axis_index("c")
    base = wid * b_per_w

    pltpu.sync_copy(heads.at[pl.ds(base, b_per_w)], hidx)
    pltpu.sync_copy(rels.at[pl.ds(base, b_per_w)], ridx)
    pltpu.sync_copy(tails.at[pl.ds(base, b_per_w)], tidx)
    pltpu.sync_copy(wsplit, wv)

    cp_h = pltpu.async_copy(etab.at[hidx], hrow, sem_h)
    cp_r = pltpu.async_copy(rtab.at[ridx], rrow, sem_r)
    cp_t = pltpu.async_copy(etab.at[tidx], trow, sem_t)
    cp_h.wait()
    cp_r.wait()
    cp_t.wait()

    wregs = [wv[i] for i in range(4)]
    bvec = wv[4]

    def triple_body(j, carry):
        acc_abs = jnp.zeros((LANES,), jnp.float32)
        acc_dot = jnp.zeros((LANES,), jnp.float32)
        for c in range(4):
            hv = hrow[j, pl.ds(LANES * c, LANES)]
            rv = rrow[j, pl.ds(LANES * c, LANES)]
            tv = trow[j, pl.ds(LANES * c, LANES)]
            z = (hv + rv) - tv
            acc_abs = acc_abs + jnp.abs(z)
            acc_dot = acc_dot + z * wregs[c]
        absb[j] = plsc.cumsum(acc_abs)
        dotb[j] = plsc.cumsum(acc_dot)
        return carry

    lax.fori_loop(0, b_per_w, triple_body, 0)

    riota = lax.iota(jnp.int32, LANES)
    col15 = jnp.full((LANES,), LANES - 1, jnp.int32)

    def group_body(g, carry):
        gbase = g * LANES
        rows = riota + gbase
        sa = plsc.load_gather(absb, [rows, col15])
        sd = plsc.load_gather(dotb, [rows, col15]) + bvec
        dcoef = 1.0 / (1.0 + jnp.exp(-sd))
        outv[pl.ds(gbase, LANES)] = GAMMA - dcoef * sa
        return carry

    lax.fori_loop(0, b_per_w // LANES, group_body, 0)

    pltpu.sync_copy(outv, out.at[pl.ds(base, b_per_w)])


def _run(heads, rels, tails, etab, rtab, wsplit):
    batch = heads.shape[0]
    b_per_w = batch // NW
    mesh = plsc.VectorSubcoreMesh(core_axis_name="c", subcore_axis_name="s")
    kern = functools.partial(
        pl.kernel,
        out_type=jax.ShapeDtypeStruct((batch,), jnp.float32),
        mesh=mesh,
        compiler_params=pltpu.CompilerParams(
            needs_layout_passes=False, use_tc_tiling_on_sc=False),
        scratch_types=[
            pltpu.VMEM((b_per_w,), jnp.int32),
            pltpu.VMEM((b_per_w,), jnp.int32),
            pltpu.VMEM((b_per_w,), jnp.int32),
            pltpu.VMEM((b_per_w, HIDDEN), jnp.float32),
            pltpu.VMEM((b_per_w, HIDDEN), jnp.float32),
            pltpu.VMEM((b_per_w, HIDDEN), jnp.float32),
            pltpu.VMEM((5, LANES), jnp.float32),
            pltpu.VMEM((b_per_w, LANES), jnp.float32),
            pltpu.VMEM((b_per_w, LANES), jnp.float32),
            pltpu.VMEM((b_per_w,), jnp.float32),
            pltpu.SemaphoreType.DMA,
            pltpu.SemaphoreType.DMA,
            pltpu.SemaphoreType.DMA,
        ],
    )(functools.partial(_sc_body, b_per_w=b_per_w))
    return kern(heads, rels, tails, etab, rtab, wsplit)


def kernel(sample, entity_embedding, relation_embedding, D_w, D_b):
    heads = sample[:, 0]
    rels = sample[:, 1]
    tails = sample[:, 2]
    etab = _tc_transpose(entity_embedding.T, entity_embedding.shape[0], 8192)
    rtab = _tc_transpose(relation_embedding.T, relation_embedding.shape[0], 1000)
    wsplit = jnp.concatenate(
        [D_w[:, 0].reshape(4, LANES), jnp.broadcast_to(D_b, (1, LANES))],
        axis=0)
    out = _run(heads, rels, tails, etab, rtab, wsplit)
    return out[:, None]


# speedup vs baseline: 1.3370x; 1.0929x over previous
import functools

import jax
import jax.numpy as jnp
from jax import lax
from jax.experimental import pallas as pl
from jax.experimental.pallas import tpu as pltpu
from jax.experimental.pallas import tpu_sc as plsc

GAMMA = 12.0
HIDDEN = 64
LANES = 16
NC = 2
NS = 16
NW = NC * NS


def _tc_txp_body(x_ref, o_ref):
    o_ref[...] = x_ref[...].T


def _tc_transpose(table_t, n_rows, blk):
    grid = (n_rows + blk - 1) // blk
    return pl.pallas_call(
        _tc_txp_body,
        grid=(grid,),
        in_specs=[
            pl.BlockSpec((HIDDEN, blk), lambda i: (0, i)),
        ],
        out_specs=pl.BlockSpec((blk, HIDDEN), lambda i: (i, 0)),
        out_shape=jax.ShapeDtypeStruct((n_rows, HIDDEN), jnp.float32),
    )(table_t)


def _sc_body(heads, rels, tails, etab, rtab, wsplit, out,
             hidx, ridx, tidx, hrow, rrow, trow, wv,
             absb, dotb, outv, sem_h, sem_r, sem_t, b_per_w):
    wid = lax.axis_index("s") * NC + lax.axis_index("c")
    base = wid * b_per_w

    pltpu.sync_copy(heads.at[pl.ds(base, b_per_w)], hidx)
    pltpu.sync_copy(rels.at[pl.ds(base, b_per_w)], ridx)
    pltpu.sync_copy(tails.at[pl.ds(base, b_per_w)], tidx)
    pltpu.sync_copy(wsplit, wv)

    cp_h = pltpu.async_copy(etab.at[hidx], hrow, sem_h)
    cp_r = pltpu.async_copy(rtab.at[ridx], rrow, sem_r)
    cp_t = pltpu.async_copy(etab.at[tidx], trow, sem_t)
    cp_h.wait()
    cp_r.wait()
    cp_t.wait()

    wregs = [wv[i] for i in range(4)]
    bvec = wv[4]

    def triple_body(j, carry):
        acc_abs = jnp.zeros((LANES,), jnp.float32)
        acc_dot = jnp.zeros((LANES,), jnp.float32)
        for c in range(4):
            hv = hrow[j, pl.ds(LANES * c, LANES)]
            rv = rrow[j, pl.ds(LANES * c, LANES)]
            tv = trow[j, pl.ds(LANES * c, LANES)]
            z = (hv + rv) - tv
            acc_abs = acc_abs + jnp.abs(z)
            acc_dot = acc_dot + z * wregs[c]
        absb[j] = plsc.cumsum(acc_abs)
        dotb[j] = plsc.cumsum(acc_dot)
        return carry

    lax.fori_loop(0, b_per_w, triple_body, 0)

    riota = lax.iota(jnp.int32, LANES)
    col15 = jnp.full((LANES,), LANES - 1, jnp.int32)

    def group_body(g, carry):
        gbase = g * LANES
        rows = riota + gbase
        sa = plsc.load_gather(absb, [rows, col15])
        sd = plsc.load_gather(dotb, [rows, col15]) + bvec
        dcoef = 1.0 / (1.0 + jnp.exp(-sd))
        outv[pl.ds(gbase, LANES)] = GAMMA - dcoef * sa
        return carry

    lax.fori_loop(0, b_per_w // LANES, group_body, 0)

    pltpu.sync_copy(outv, out.at[pl.ds(base, b_per_w)])


def _run(heads, rels, tails, etab, rtab, wsplit):
    batch = heads.shape[0]
    b_per_w = batch // NW
    mesh = plsc.VectorSubcoreMesh(core_axis_name="c", subcore_axis_name="s")
    kern = functools.partial(
        pl.kernel,
        out_type=jax.ShapeDtypeStruct((batch,), jnp.float32),
        mesh=mesh,
        compiler_params=pltpu.CompilerParams(
            needs_layout_passes=False, use_tc_tiling_on_sc=False),
        scratch_types=[
            pltpu.VMEM((b_per_w,), jnp.int32),
            pltpu.VMEM((b_per_w,), jnp.int32),
            pltpu.VMEM((b_per_w,), jnp.int32),
            pltpu.VMEM((b_per_w, HIDDEN), jnp.float32),
            pltpu.VMEM((b_per_w, HIDDEN), jnp.float32),
            pltpu.VMEM((b_per_w, HIDDEN), jnp.float32),
            pltpu.VMEM((5, LANES), jnp.float32),
            pltpu.VMEM((b_per_w, LANES), jnp.float32),
            pltpu.VMEM((b_per_w, LANES), jnp.float32),
            pltpu.VMEM((b_per_w,), jnp.float32),
            pltpu.SemaphoreType.DMA,
            pltpu.SemaphoreType.DMA,
            pltpu.SemaphoreType.DMA,
        ],
    )(functools.partial(_sc_body, b_per_w=b_per_w))
    return kern(heads, rels, tails, etab, rtab, wsplit)


def kernel(sample, entity_embedding, relation_embedding, D_w, D_b):
    heads = sample[:, 0]
    rels = sample[:, 1]
    tails = sample[:, 2]
    etab = entity_embedding
    rtab = relation_embedding
    wsplit = jnp.concatenate(
        [D_w[:, 0].reshape(4, LANES), jnp.broadcast_to(D_b, (1, LANES))],
        axis=0)
    out = _run(heads, rels, tails, etab, rtab, wsplit)
    return out[:, None]
